# baseline (device time: 19638 ns/iter reference)
import jax
import jax.numpy as jnp
from jax import lax
from jax.experimental import pallas as pl
from jax.experimental.pallas import tpu as pltpu

B, H, D = 8, 8, 64
KLOC = 512
SCALE = D ** -0.5


def kernel(Q, K, V):
    Q2 = Q.reshape(B, H, D)
    K2 = K.reshape(B, KLOC, H * D)
    V2 = V.reshape(B, KLOC, H * D)

    def body(q_ref, k_ref, v_ref, o_ref,
             acc, stats, peer_acc, peer_stats, send_sems, recv_sems):
        my_x = lax.axis_index("x")
        my_y = lax.axis_index("y")
        my_z = lax.axis_index("z")
        partner = (1 - my_x, my_y, my_z)

        colh = lax.broadcasted_iota(jnp.int32, (H, H * D), 1) // D
        rowh = lax.broadcasted_iota(jnp.int32, (H, H * D), 0)
        qmask = (colh == rowh).astype(jnp.float32)
        eye3 = (lax.broadcasted_iota(jnp.int32, (H, H, 1), 0)
                == lax.broadcasted_iota(jnp.int32, (H, H, 1), 1)
                ).astype(jnp.float32)

        ms, ls, os_ = [], [], []
        for b in range(B):
            qb = q_ref[b]
            qblk = jnp.concatenate([qb] * H, axis=1) * qmask
            sblks = []
            for j in range(H * D // 128):
                kb_j = k_ref[b, :, 128 * j:128 * (j + 1)]
                qb_j = qblk[2 * j:2 * j + 2, 128 * j:128 * (j + 1)]
                sblks.append(lax.dot_general(
                    qb_j, kb_j, (((1,), (1,)), ((), ())),
                    preferred_element_type=jnp.float32))
            s = jnp.concatenate(sblks, axis=0) * SCALE
            m = jnp.max(s, axis=1, keepdims=True)
            p = jnp.exp(s - m)
            l = jnp.sum(p, axis=1, keepdims=True)
            t = lax.dot_general(
                p, v_ref[b], (((1,), (0,)), ((), ())),
                preferred_element_type=jnp.float32)
            ob = jnp.sum(t.reshape(H, H, D) * eye3, axis=0)
            ms.append(m.reshape(1, H))
            ls.append(l.reshape(1, H))
            os_.append(ob)
        acc[...] = jnp.stack(os_, axis=0)
        stats[0] = jnp.concatenate(ms, axis=0)
        stats[1] = jnp.concatenate(ls, axis=0)

        barrier_sem = pltpu.get_barrier_semaphore()
        pl.semaphore_signal(barrier_sem, inc=1, device_id=partner,
                            device_id_type=pl.DeviceIdType.MESH)
        pl.semaphore_wait(barrier_sem, 1)

        rdma_o = pltpu.make_async_remote_copy(
            src_ref=acc, dst_ref=peer_acc,
            send_sem=send_sems.at[0], recv_sem=recv_sems.at[0],
            device_id=partner, device_id_type=pl.DeviceIdType.MESH)
        rdma_s = pltpu.make_async_remote_copy(
            src_ref=stats, dst_ref=peer_stats,
            send_sem=send_sems.at[1], recv_sem=recv_sems.at[1],
            device_id=partner, device_id_type=pl.DeviceIdType.MESH)
        rdma_o.start()
        rdma_s.start()
        rdma_o.wait()
        rdma_s.wait()

        m_s, l_s = stats[0], stats[1]
        m_p, l_p = peer_stats[0], peer_stats[1]
        m_n = jnp.maximum(m_s, m_p)
        a_s = jnp.exp(m_s - m_n)
        a_p = jnp.exp(m_p - m_n)
        l_n = a_s * l_s + a_p * l_p
        o = (a_s[:, :, None] * acc[...] + a_p[:, :, None] * peer_acc[...]) \
            / l_n[:, :, None]
        o_ref[...] = o[:, None]

    return pl.pallas_call(
        body,
        out_shape=jax.ShapeDtypeStruct((B, 1, H, D), jnp.float32),
        in_specs=[
            pl.BlockSpec(memory_space=pltpu.VMEM),
            pl.BlockSpec(memory_space=pltpu.VMEM),
            pl.BlockSpec(memory_space=pltpu.VMEM),
        ],
        out_specs=pl.BlockSpec(memory_space=pltpu.VMEM),
        scratch_shapes=[
            pltpu.VMEM((B, H, D), jnp.float32),
            pltpu.VMEM((2, B, H), jnp.float32),
            pltpu.VMEM((B, H, D), jnp.float32),
            pltpu.VMEM((2, B, H), jnp.float32),
            pltpu.SemaphoreType.DMA((2,)),
            pltpu.SemaphoreType.DMA((2,)),
        ],
        compiler_params=pltpu.CompilerParams(collective_id=0),
    )(Q2, K2, V2)


# device time: 18572 ns/iter; 1.0574x vs baseline; 1.0574x over previous
import jax
import jax.numpy as jnp
from jax import lax
from jax.experimental import pallas as pl
from jax.experimental.pallas import tpu as pltpu

B, H, D = 8, 8, 64
KLOC = 512
NYZ = 8
KSUB = KLOC // NYZ
NDEV = 16
SCALE = D ** -0.5

_POSITIONS = [(qx, qy, qz, (qx * 2 + qy) * 4 + qz)
              for qx in range(2) for qy in range(2) for qz in range(4)]


def kernel(Q, K, V):
    Q2 = Q.reshape(B, H, D)
    K2 = K.reshape(B, KLOC, H * D)
    V2 = V.reshape(B, KLOC, H * D)

    my_yz = lax.axis_index("y") * 4 + lax.axis_index("z")
    start = my_yz * KSUB
    KsT = lax.dynamic_slice_in_dim(K2, start, KSUB, axis=1).transpose(0, 2, 1)
    Vs = lax.dynamic_slice_in_dim(V2, start, KSUB, axis=1)

    def body(q_ref, k_ref, v_ref, o_ref,
             comm_o, comm_s, so_sems, ss_sems, ro_sems, rs_sems):
        my_x = lax.axis_index("x")
        my_y = lax.axis_index("y")
        my_z = lax.axis_index("z")
        my_lin = (my_x * 2 + my_y) * 4 + my_z

        colh = lax.broadcasted_iota(jnp.int32, (H, H * D), 1) // D
        rowh = lax.broadcasted_iota(jnp.int32, (H, H * D), 0)
        qmask = (colh == rowh).astype(jnp.float32)
        eye3 = (lax.broadcasted_iota(jnp.int32, (H, H, 1), 0)
                == lax.broadcasted_iota(jnp.int32, (H, H, 1), 1)
                ).astype(jnp.float32)

        ms, ls, os_ = [], [], []
        for b in range(B):
            qb = q_ref[b]
            qblk = jnp.concatenate([qb] * H, axis=1) * qmask
            s = lax.dot_general(
                qblk, k_ref[b], (((1,), (0,)), ((), ())),
                preferred_element_type=jnp.float32) * SCALE
            m = jnp.max(s, axis=1, keepdims=True)
            p = jnp.exp(s - m)
            l = jnp.sum(p, axis=1, keepdims=True)
            t = lax.dot_general(
                p, v_ref[b], (((1,), (0,)), ((), ())),
                preferred_element_type=jnp.float32)
            ob = jnp.sum(t.reshape(H, H, D) * eye3, axis=0)
            ms.append(m.reshape(1, H))
            ls.append(l.reshape(1, H))
            os_.append(ob)
        comm_o[my_lin] = jnp.stack(os_, axis=0)
        comm_s[my_lin] = jnp.stack(
            [jnp.concatenate(ms, axis=0), jnp.concatenate(ls, axis=0)],
            axis=0)

        bar = pltpu.get_barrier_semaphore()
        for qx, qy, qz, lin_q in _POSITIONS:
            @pl.when(lin_q != my_lin)
            def _(qx=qx, qy=qy, qz=qz):
                pl.semaphore_signal(bar, inc=1, device_id=(qx, qy, qz),
                                    device_id_type=pl.DeviceIdType.MESH)
        pl.semaphore_wait(bar, NDEV - 1)

        def out_descs(qx, qy, qz, lin_q):
            ro = pltpu.make_async_remote_copy(
                src_ref=comm_o.at[my_lin], dst_ref=comm_o.at[my_lin],
                send_sem=so_sems.at[lin_q], recv_sem=ro_sems.at[my_lin],
                device_id=(qx, qy, qz), device_id_type=pl.DeviceIdType.MESH)
            rs = pltpu.make_async_remote_copy(
                src_ref=comm_s.at[my_lin], dst_ref=comm_s.at[my_lin],
                send_sem=ss_sems.at[lin_q], recv_sem=rs_sems.at[my_lin],
                device_id=(qx, qy, qz), device_id_type=pl.DeviceIdType.MESH)
            return ro, rs

        for qx, qy, qz, lin_q in _POSITIONS:
            @pl.when(lin_q != my_lin)
            def _(qx=qx, qy=qy, qz=qz, lin_q=lin_q):
                ro, rs = out_descs(qx, qy, qz, lin_q)
                ro.start()
                rs.start()

        for qx, qy, qz, lin_q in _POSITIONS:
            @pl.when(lin_q != my_lin)
            def _(qx=qx, qy=qy, qz=qz, lin_q=lin_q):
                rco = pltpu.make_async_remote_copy(
                    src_ref=comm_o.at[lin_q], dst_ref=comm_o.at[lin_q],
                    send_sem=so_sems.at[lin_q], recv_sem=ro_sems.at[lin_q],
                    device_id=(qx, qy, qz),
                    device_id_type=pl.DeviceIdType.MESH)
                rcs = pltpu.make_async_remote_copy(
                    src_ref=comm_s.at[lin_q], dst_ref=comm_s.at[lin_q],
                    send_sem=ss_sems.at[lin_q], recv_sem=rs_sems.at[lin_q],
                    device_id=(qx, qy, qz),
                    device_id_type=pl.DeviceIdType.MESH)
                rco.wait_recv()
                rcs.wait_recv()

        m_all = comm_s[:, 0]
        l_all = comm_s[:, 1]
        m_n = jnp.max(m_all, axis=0)
        w = jnp.exp(m_all - m_n[None])
        l_n = jnp.sum(w * l_all, axis=0)
        o = jnp.sum(w[..., None] * comm_o[...], axis=0) \
            / l_n[..., None]
        o_ref[...] = o[:, None]

        for qx, qy, qz, lin_q in _POSITIONS:
            @pl.when(lin_q != my_lin)
            def _(qx=qx, qy=qy, qz=qz, lin_q=lin_q):
                ro, rs = out_descs(qx, qy, qz, lin_q)
                ro.wait_send()
                rs.wait_send()

    return pl.pallas_call(
        body,
        out_shape=jax.ShapeDtypeStruct((B, 1, H, D), jnp.float32),
        in_specs=[
            pl.BlockSpec(memory_space=pltpu.VMEM),
            pl.BlockSpec(memory_space=pltpu.VMEM),
            pl.BlockSpec(memory_space=pltpu.VMEM),
        ],
        out_specs=pl.BlockSpec(memory_space=pltpu.VMEM),
        scratch_shapes=[
            pltpu.VMEM((NDEV, B, H, D), jnp.float32),
            pltpu.VMEM((NDEV, 2, B, H), jnp.float32),
            pltpu.SemaphoreType.DMA((NDEV,)),
            pltpu.SemaphoreType.DMA((NDEV,)),
            pltpu.SemaphoreType.DMA((NDEV,)),
            pltpu.SemaphoreType.DMA((NDEV,)),
        ],
        compiler_params=pltpu.CompilerParams(collective_id=0),
    )(Q2, KsT, Vs)
